# baseline (speedup 1.0000x reference)
import functools

import jax
import jax.numpy as jnp
from jax import lax
from jax.experimental import pallas as pl
from jax.experimental.pallas import tpu as pltpu
from jax.experimental.pallas import tpu_sc as plsc

DIM = 16


def _dyn_gather(v, idx):
    dnums = lax.GatherDimensionNumbers(
        offset_dims=(), collapsed_slice_dims=(0,), start_index_map=(0,)
    )
    return lax.gather(
        v,
        idx[:, None],
        dnums,
        slice_sizes=(1,),
        mode=lax.GatherScatterMode.PROMISE_IN_BOUNDS,
    )


_info = plsc.get_sparse_core_info()
_NC, _NS, _NL = _info.num_cores, _info.num_subcores, _info.num_lanes
_NW = _NC * _NS


@functools.lru_cache(maxsize=None)
def _build(batch: int, hist: int, vocab: int, rows_per_chunk: int, sub: int):
    n_idx = batch * hist
    per_w = n_idx // _NW
    chunk = rows_per_chunk * hist
    n_chunks = per_w // chunk
    n_sub = chunk // sub
    n_fire = sub // 128
    n_groups = sub // _NL

    mesh = plsc.VectorSubcoreMesh(core_axis_name="c", subcore_axis_name="s")

    @functools.partial(
        pl.kernel,
        mesh=mesh,
        compiler_params=pltpu.CompilerParams(use_tc_tiling_on_sc=False),
        out_type=jax.ShapeDtypeStruct((batch, hist, DIM), jnp.float32),
        scratch_types=[
            pltpu.VMEM((chunk,), jnp.int32),
            pltpu.VMEM((chunk,), jnp.int32),
            pltpu.VMEM((sub, 2 * DIM), jnp.float32),
            pltpu.VMEM((rows_per_chunk, hist, DIM), jnp.float32),
            pltpu.SemaphoreType.DMA,
        ],
    )
    def k(x_hbm, w_hbm, out_hbm, idx_v, pidx_v, rows_v, out_v, sem):
        wid = lax.axis_index("s") * _NC + lax.axis_index("c")
        iota = lax.iota(jnp.int32, _NL)

        def chunk_body(ci, carry):
            base = wid * per_w + ci * chunk
            pltpu.sync_copy(x_hbm.at[pl.ds(base, chunk)], idx_v)

            def pidx_body(g, c0):
                b0 = g * _NL
                pidx_v[pl.ds(b0, _NL)] = idx_v[pl.ds(b0, _NL)] >> 1
                return c0

            lax.fori_loop(0, chunk // _NL, pidx_body, 0)

            def sub_body(si, c1):
                s0 = si * sub
                cps = [
                    pltpu.async_copy(
                        w_hbm.at[pidx_v.at[pl.ds(s0 + j * 128, 128)]],
                        rows_v.at[pl.ds(j * 128, 128)],
                        sem,
                    )
                    for j in range(n_fire)
                ]
                for cp in cps:
                    cp.wait()

                def group_body(g, c2):
                    b0 = g * _NL
                    idx16 = idx_v[pl.ds(s0 + b0, _NL)]
                    for r in range(_NL):
                        q = s0 + b0 + r
                        i = q // hist
                        j = q - i * hist
                        half = (idx16[r] & 1) * DIM
                        row = rows_v[b0 + r, pl.ds(half, DIM)]
                        e = jnp.exp(row)
                        t = e
                        for sh in (1, 2, 4, 8):
                            t = t + _dyn_gather(t, iota ^ sh)
                        flag = jnp.where(idx16[r] == 0, jnp.float32(0.0),
                                         jnp.float32(1.0))
                        out_v[i, j] = e * (flag / t)
                    return c2

                lax.fori_loop(0, n_groups, group_body, 0)
                return c1

            lax.fori_loop(0, n_sub, sub_body, 0)
            pltpu.sync_copy(
                out_v,
                out_hbm.at[pl.ds(wid * (per_w // hist) + ci * rows_per_chunk,
                                 rows_per_chunk)],
            )
            return carry

        lax.fori_loop(0, n_chunks, chunk_body, 0)

    return k


def kernel(x, weight):
    b, h = x.shape
    vocab, dim = weight.shape
    w2 = weight.reshape(vocab // 2, 2 * dim)
    return _build(b, h, vocab, 64, 640)(x.reshape(b * h), w2)


# speedup vs baseline: 1.4600x; 1.4600x over previous
import functools

import jax
import jax.numpy as jnp
from jax import lax
from jax.experimental import pallas as pl
from jax.experimental.pallas import tpu as pltpu
from jax.experimental.pallas import tpu_sc as plsc

DIM = 16


def _dyn_gather(v, idx):
    dnums = lax.GatherDimensionNumbers(
        offset_dims=(), collapsed_slice_dims=(0,), start_index_map=(0,)
    )
    return lax.gather(
        v,
        idx[:, None],
        dnums,
        slice_sizes=(1,),
        mode=lax.GatherScatterMode.PROMISE_IN_BOUNDS,
    )


_info = plsc.get_sparse_core_info()
_NC, _NS, _NL = _info.num_cores, _info.num_subcores, _info.num_lanes
_NW = _NC * _NS


@functools.lru_cache(maxsize=None)
def _build(batch: int, hist: int, vocab: int, rows_per_chunk: int):
    n_idx = batch * hist
    per_w = n_idx // _NW
    chunk = rows_per_chunk * hist
    n_chunks = per_w // chunk
    n_fire = chunk // 128
    n_groups = chunk // _NL

    mesh = plsc.VectorSubcoreMesh(core_axis_name="c", subcore_axis_name="s")

    @functools.partial(
        pl.kernel,
        mesh=mesh,
        compiler_params=pltpu.CompilerParams(use_tc_tiling_on_sc=False),
        out_type=jax.ShapeDtypeStruct((batch, hist, DIM), jnp.float32),
        scratch_types=[
            pltpu.VMEM((chunk,), jnp.int32),
            pltpu.VMEM((chunk, DIM), jnp.float32),
            pltpu.VMEM((rows_per_chunk, hist, DIM), jnp.float32),
            pltpu.SemaphoreType.DMA,
        ],
    )
    def k(x_hbm, w_hbm, out_hbm, idx_v, rows_v, out_v, sem):
        wid = lax.axis_index("s") * _NC + lax.axis_index("c")
        iota = lax.iota(jnp.int32, _NL)

        def chunk_body(ci, carry):
            base = wid * per_w + ci * chunk
            pltpu.sync_copy(x_hbm.at[pl.ds(base, chunk)], idx_v)
            cps = [
                pltpu.async_copy(
                    w_hbm.at[idx_v.at[pl.ds(j * 128, 128)]],
                    rows_v.at[pl.ds(j * 128, 128)],
                    sem,
                )
                for j in range(n_fire)
            ]
            for cp in cps:
                cp.wait()

            def group_body(g, carry2):
                b0 = g * _NL
                idx16 = idx_v[pl.ds(b0, _NL)]
                for r in range(_NL):
                    row = rows_v[b0 + r]
                    e = jnp.exp(row)
                    s = e
                    for sh in (1, 2, 4, 8):
                        s = s + _dyn_gather(s, iota ^ sh)
                    flag = jnp.where(idx16[r] == 0, jnp.float32(0.0),
                                     jnp.float32(1.0))
                    rows_v[b0 + r] = e * (flag / s)
                return carry2

            lax.fori_loop(0, n_groups, group_body, 0)

            def repack_body(i, carry3):
                def inner(j, carry4):
                    out_v[i, j] = rows_v[i * hist + j]
                    return carry4

                lax.fori_loop(0, hist, inner, 0)
                return carry3

            lax.fori_loop(0, rows_per_chunk, repack_body, 0)
            pltpu.sync_copy(
                out_v,
                out_hbm.at[pl.ds(wid * (per_w // hist) + ci * rows_per_chunk,
                                 rows_per_chunk)],
            )
            return carry

        lax.fori_loop(0, n_chunks, chunk_body, 0)

    return k


def kernel(x, weight):
    b, h = x.shape
    vocab, dim = weight.shape
    return _build(b, h, vocab, 64)(x.reshape(b * h), weight)


# speedup vs baseline: 2.1334x; 1.4613x over previous
import functools

import jax
import jax.numpy as jnp
from jax import lax
from jax.experimental import pallas as pl
from jax.experimental.pallas import tpu as pltpu
from jax.experimental.pallas import tpu_sc as plsc

DIM = 16


def _dyn_gather(v, idx):
    dnums = lax.GatherDimensionNumbers(
        offset_dims=(), collapsed_slice_dims=(0,), start_index_map=(0,)
    )
    return lax.gather(
        v,
        idx[:, None],
        dnums,
        slice_sizes=(1,),
        mode=lax.GatherScatterMode.PROMISE_IN_BOUNDS,
    )


def _transpose16(vs, iota):
    vs = list(vs)
    for s in (1, 2, 4, 8):
        perm = iota ^ s
        mask = (iota & s) == 0
        nxt = list(vs)
        for i in range(DIM):
            if i & s == 0:
                a, b = vs[i], vs[i | s]
                nxt[i] = jnp.where(mask, a, _dyn_gather(b, perm))
                nxt[i | s] = jnp.where(mask, _dyn_gather(a, perm), b)
        vs = nxt
    return vs


_info = plsc.get_sparse_core_info()
_NC, _NS, _NL = _info.num_cores, _info.num_subcores, _info.num_lanes
_NW = _NC * _NS


@functools.lru_cache(maxsize=None)
def _build(batch: int, hist: int, vocab: int):
    bpw = batch // _NW
    n_fire = bpw // 128
    n_groups = bpw // _NL

    mesh = plsc.VectorSubcoreMesh(core_axis_name="c", subcore_axis_name="s")

    @functools.partial(
        pl.kernel,
        mesh=mesh,
        compiler_params=pltpu.CompilerParams(use_tc_tiling_on_sc=True),
        out_type=jax.ShapeDtypeStruct((hist, DIM, batch), jnp.float32),
        scratch_types=[
            pltpu.VMEM((bpw,), jnp.int32),
            pltpu.VMEM((bpw,), jnp.int32),
            pltpu.VMEM((bpw, 8 * DIM), jnp.float32),
            pltpu.VMEM((DIM, bpw), jnp.float32),
            pltpu.SemaphoreType.DMA,
        ],
    )
    def k(xt_hbm, w_hbm, out_hbm, idx_v, pidx_v, rows_v, out_v, sem):
        wid = lax.axis_index("s") * _NC + lax.axis_index("c")
        b0 = wid * bpw
        iota = lax.iota(jnp.int32, _NL)

        def h_body(h, carry):
            pltpu.sync_copy(xt_hbm.at[h, pl.ds(b0, bpw)], idx_v)

            def pidx_body(g, c0):
                sl = pl.ds(g * _NL, _NL)
                pidx_v[sl] = idx_v[sl] >> 3
                return c0

            lax.fori_loop(0, n_groups, pidx_body, 0)
            cps = [
                pltpu.async_copy(
                    w_hbm.at[pidx_v.at[pl.ds(j * 128, 128)]],
                    rows_v.at[pl.ds(j * 128, 128)],
                    sem,
                )
                for j in range(n_fire)
            ]
            for cp in cps:
                cp.wait()

            def group_body(g, c1):
                gb = g * _NL
                idx16 = idx_v[pl.ds(gb, _NL)]
                es = []
                for r in range(_NL):
                    sub = (idx16[r] & 7) * DIM
                    row = rows_v[gb + r, pl.ds(sub, DIM)]
                    es.append(jnp.exp(row))
                cs = _transpose16(es, iota)
                t = cs[0]
                for v in cs[1:]:
                    t = t + v
                scale = jnp.where(idx16 == 0, jnp.float32(0.0), 1.0 / t)
                for d in range(DIM):
                    out_v[d, pl.ds(gb, _NL)] = cs[d] * scale
                return c1

            lax.fori_loop(0, n_groups, group_body, 0)
            pltpu.sync_copy(out_v, out_hbm.at[h, :, pl.ds(b0, bpw)])
            return carry

        lax.fori_loop(0, hist, h_body, 0)

    return k


def kernel(x, weight):
    b, h = x.shape
    vocab, dim = weight.shape
    w128 = weight.reshape(vocab // 8, 8 * dim)
    yt = _build(b, h, vocab)(x.T, w128)
    return jnp.transpose(yt, (2, 0, 1))


# speedup vs baseline: 2.9361x; 1.3762x over previous
import functools

import jax
import jax.numpy as jnp
from jax import lax
from jax.experimental import pallas as pl
from jax.experimental.pallas import tpu as pltpu
from jax.experimental.pallas import tpu_sc as plsc

DIM = 16


def _dyn_gather(v, idx):
    dnums = lax.GatherDimensionNumbers(
        offset_dims=(), collapsed_slice_dims=(0,), start_index_map=(0,)
    )
    return lax.gather(
        v,
        idx[:, None],
        dnums,
        slice_sizes=(1,),
        mode=lax.GatherScatterMode.PROMISE_IN_BOUNDS,
    )


def _transpose16(vs, iota):
    vs = list(vs)
    for s in (1, 2, 4, 8):
        perm = iota ^ s
        mask = (iota & s) == 0
        nxt = list(vs)
        for i in range(DIM):
            if i & s == 0:
                a, b = vs[i], vs[i | s]
                nxt[i] = jnp.where(mask, a, _dyn_gather(b, perm))
                nxt[i | s] = jnp.where(mask, _dyn_gather(a, perm), b)
        vs = nxt
    return vs


_info = plsc.get_sparse_core_info()
_NC, _NS, _NL = _info.num_cores, _info.num_subcores, _info.num_lanes
_NW = _NC * _NS


@functools.lru_cache(maxsize=None)
def _build(batch: int, hist: int, vocab: int):
    bpw = batch // _NW

    mesh = plsc.VectorSubcoreMesh(core_axis_name="c", subcore_axis_name="s")

    half = bpw // 2
    n_fire_h = half // 128
    n_hgroups = half // _NL

    @functools.partial(
        pl.kernel,
        mesh=mesh,
        compiler_params=pltpu.CompilerParams(use_tc_tiling_on_sc=True),
        out_type=jax.ShapeDtypeStruct((hist, DIM, batch), jnp.float32),
        scratch_types=[
            pltpu.VMEM((half,), jnp.int32),
            pltpu.VMEM((half,), jnp.int32),
            pltpu.VMEM((half,), jnp.int32),
            pltpu.VMEM((half,), jnp.int32),
            pltpu.VMEM((half, 8 * DIM), jnp.float32),
            pltpu.VMEM((half, 8 * DIM), jnp.float32),
            pltpu.VMEM((DIM, bpw), jnp.float32),
            pltpu.SemaphoreType.DMA,
            pltpu.SemaphoreType.DMA,
        ],
    )
    def k(xt_hbm, w_hbm, out_hbm, idx0, idx1, pidx0, pidx1, rows0, rows1,
          out_v, sem0, sem1):
        idxs = (idx0, idx1)
        pidxs = (pidx0, pidx1)
        rows = (rows0, rows1)
        sems = (sem0, sem1)
        wid = lax.axis_index("s") * _NC + lax.axis_index("c")
        b0 = wid * bpw
        iota = lax.iota(jnp.int32, _NL)

        def prep(h, hb, buf):
            pltpu.sync_copy(xt_hbm.at[h, pl.ds(b0 + hb * half, half)],
                            idxs[buf])

            def pidx_body(g, c0):
                sl = pl.ds(g * _NL, _NL)
                pidxs[buf][sl] = idxs[buf][sl] >> 3
                return c0

            lax.fori_loop(0, n_hgroups, pidx_body, 0)
            for j in range(n_fire_h):
                pltpu.async_copy(
                    w_hbm.at[pidxs[buf].at[pl.ds(j * 128, 128)]],
                    rows[buf].at[pl.ds(j * 128, 128)],
                    sems[buf],
                )

        prep(0, 0, 0)

        def h_body(h, carry):
            for b in (0, 1):
                for j in range(n_fire_h):
                    pltpu.make_async_copy(
                        w_hbm.at[pidxs[b].at[pl.ds(j * 128, 128)]],
                        rows[b].at[pl.ds(j * 128, 128)],
                        sems[b],
                    ).wait()
                if b == 0:
                    prep(h, 1, 1)
                else:
                    @pl.when(h + 1 < hist)
                    def _():
                        prep(h + 1, 0, 0)

                def group_body(g, c1):
                    gb = g * _NL
                    idx16 = idxs[b][pl.ds(gb, _NL)]
                    es = []
                    for r in range(_NL):
                        sub = (idx16[r] & 7) * DIM
                        es.append(rows[b][gb + r, pl.ds(sub, DIM)])
                    cs = _transpose16(es, iota)
                    pad = idx16 == 0
                    zero = jnp.zeros((_NL,), jnp.float32)
                    for d in range(DIM):
                        out_v[d, pl.ds(b * half + gb, _NL)] = jnp.where(
                            pad, zero, cs[d])
                    return c1

                lax.fori_loop(0, n_hgroups, group_body, 0)
            pltpu.sync_copy(out_v, out_hbm.at[h, :, pl.ds(b0, bpw)])
            return carry

        lax.fori_loop(0, hist, h_body, 0)

    return k


def _repack_body(wt_ref, out_ref):
    blk = wt_ref[...]
    e = jnp.exp(blk)
    y = e / jnp.sum(e, axis=0, keepdims=True)
    v = y.T
    v3 = v.reshape(v.shape[0] // 8, 8, 16)
    for k in range(8):
        out_ref[:, k * 16:(k + 1) * 16] = v3[:, k, :]


@functools.lru_cache(maxsize=None)
def _build_repack(vocab: int, dim: int, bk: int):
    grid = pl.cdiv(vocab, bk)
    return pl.pallas_call(
        _repack_body,
        grid=(grid,),
        in_specs=[pl.BlockSpec((dim, bk), lambda i: (0, i))],
        out_specs=pl.BlockSpec((bk // 8, 8 * dim), lambda i: (i, 0)),
        out_shape=jax.ShapeDtypeStruct((vocab // 8, 8 * dim), jnp.float32),
    )


def kernel(x, weight):
    b, h = x.shape
    vocab, dim = weight.shape
    w128 = _build_repack(vocab, dim, 8192)(weight.T)
    yt = _build(b, h, vocab)(x.T, w128)
    return jnp.transpose(yt, (2, 0, 1))
